# Initial kernel scaffold; baseline (speedup 1.0000x reference)
import jax
import jax.numpy as jnp
from jax.experimental import pallas as pl


def kernel(x, edge_index0, edge_index1, W_self0, W_neigh0, b0, W_self1, W_neigh1, b1):
    raise NotImplementedError("write your pallas kernel here")


# speedup vs baseline: 6.3011x; 6.3011x over previous
import functools

import jax
import jax.numpy as jnp
from jax import lax
from jax.experimental import pallas as pl
from jax.experimental.pallas import tpu as pltpu
from jax.experimental.pallas import tpu_sc as plsc

NC = 2
NS = 16
NW = NC * NS
CH = 128


def _round_up(v, m):
    return (v + m - 1) // m * m


@functools.lru_cache(maxsize=None)
def _make_sc_segsum(n_pad, nch, width):
    rows_per_tile = n_pad // NS
    mesh = plsc.VectorSubcoreMesh(
        core_axis_name="c", subcore_axis_name="s", num_cores=NC, num_subcores=NS
    )

    @functools.partial(
        pl.kernel,
        out_type=jax.ShapeDtypeStruct((NC, n_pad, width), jnp.float32),
        mesh=mesh,
        scratch_types=[
            pltpu.VMEM((nch, CH), jnp.int32),
            pltpu.VMEM((nch, CH), jnp.int32),
            pltpu.VMEM((CH, width), jnp.float32),
            pltpu.VMEM_SHARED((n_pad, width), jnp.float32),
            pltpu.SemaphoreType.DMA,
        ],
        compiler_params=pltpu.CompilerParams(use_tc_tiling_on_sc=False),
    )
    def sc_segsum(table_hbm, src_hbm, dst_hbm, zeros_hbm, out_hbm,
                  src_v, dst_v, rows_v, acc_sh, sem):
        cid = lax.axis_index("c")
        sid = lax.axis_index("s")
        wid = sid * NC + cid
        row0 = sid * rows_per_tile
        pltpu.sync_copy(zeros_hbm, acc_sh.at[pl.ds(row0, rows_per_tile)])
        pltpu.sync_copy(src_hbm.at[wid], src_v)
        pltpu.sync_copy(dst_hbm.at[wid], dst_v)
        plsc.subcore_barrier()

        def body(j, carry):
            pltpu.async_copy(table_hbm.at[src_v.at[j]], rows_v, sem).wait()
            pltpu.sync_copy(rows_v, acc_sh.at[dst_v.at[j]], add=True)
            return carry

        lax.fori_loop(0, nch, body, 0)
        plsc.subcore_barrier()
        pltpu.sync_copy(acc_sh.at[pl.ds(row0, rows_per_tile)],
                        out_hbm.at[cid, pl.ds(row0, rows_per_tile)])

    return sc_segsum


def _tc_mid_body(x_ref, acc_ref, ws0_ref, wn0_ref, b0_ref, wn1_ref, ws1_ref,
                 y1_ref, hs_ref, *, dcol, c1):
    a = acc_ref[0] + acc_ref[1]
    deg = jnp.clip(a[:, dcol:dcol + 1], 1.0, None)
    mean = a[:, :dcol] / deg
    h = jnp.dot(x_ref[...], ws0_ref[...], preferred_element_type=jnp.float32)
    h = h + jnp.dot(mean, wn0_ref[...], preferred_element_type=jnp.float32)
    h = jnp.maximum(h + b0_ref[...], 0.0)
    y1 = jnp.dot(h, wn1_ref[...], preferred_element_type=jnp.float32)
    col = lax.broadcasted_iota(jnp.int32, y1.shape, 1)
    y1_ref[...] = jnp.where(col == c1 - 1, 1.0, y1)
    hs_ref[...] = jnp.dot(h, ws1_ref[...], preferred_element_type=jnp.float32)


def _tc_out_body(hs_ref, acc_ref, b1_ref, o_ref, *, ccol):
    a = acc_ref[0] + acc_ref[1]
    deg = jnp.clip(a[:, ccol:ccol + 1], 1.0, None)
    o_ref[...] = hs_ref[...] + a / deg + b1_ref[...]


def _prep_edges(edge_index, e_pad, nch, dump_row):
    src = edge_index[0].astype(jnp.int32)
    dst = edge_index[1].astype(jnp.int32)
    pad = e_pad - src.shape[0]
    if pad:
        src = jnp.concatenate([src, jnp.zeros((pad,), jnp.int32)])
        dst = jnp.concatenate([dst, jnp.full((pad,), dump_row, jnp.int32)])
    return src.reshape(NW, nch, CH), dst.reshape(NW, nch, CH)


def kernel(x, edge_index0, edge_index1, W_self0, W_neigh0, b0, W_self1, W_neigh1, b1):
    n, d = x.shape
    h_dim = W_self0.shape[1]
    c = W_self1.shape[1]
    e = edge_index0.shape[1]

    w0 = d + 16
    c1 = _round_up(c + 1, 16)
    n_pad = _round_up(n + 1, 128)
    e_pad = _round_up(e, NW * CH)
    nch = e_pad // (NW * CH)
    rows_per_tile = n_pad // NS

    src0, dst0 = _prep_edges(edge_index0, e_pad, nch, n)
    src1, dst1 = _prep_edges(edge_index1, e_pad, nch, n)

    table0 = jnp.concatenate(
        [x, jnp.ones((n, 1), jnp.float32), jnp.zeros((n, 15), jnp.float32)], axis=1)
    zeros0 = jnp.zeros((rows_per_tile, w0), jnp.float32)
    zeros1 = jnp.zeros((rows_per_tile, c1), jnp.float32)

    b0r = b0.reshape(1, h_dim)
    b1p = jnp.concatenate([b1, jnp.zeros((c1 - c,), jnp.float32)]).reshape(1, c1)
    wn1p = jnp.pad(W_neigh1, ((0, 0), (0, c1 - c)))
    ws1p = jnp.pad(W_self1, ((0, 0), (0, c1 - c)))

    acc0 = _make_sc_segsum(n_pad, nch, w0)(table0, src0, dst0, zeros0)

    r = 1000
    grid = (n // r,)
    y1p, hs = pl.pallas_call(
        functools.partial(_tc_mid_body, dcol=d, c1=c1),
        grid=grid,
        in_specs=[
            pl.BlockSpec((r, d), lambda i: (i, 0)),
            pl.BlockSpec((NC, r, w0), lambda i: (0, i, 0)),
            pl.BlockSpec((d, h_dim), lambda i: (0, 0)),
            pl.BlockSpec((d, h_dim), lambda i: (0, 0)),
            pl.BlockSpec((1, h_dim), lambda i: (0, 0)),
            pl.BlockSpec((h_dim, c1), lambda i: (0, 0)),
            pl.BlockSpec((h_dim, c1), lambda i: (0, 0)),
        ],
        out_specs=[pl.BlockSpec((r, c1), lambda i: (i, 0)),
                   pl.BlockSpec((r, c1), lambda i: (i, 0))],
        out_shape=[jax.ShapeDtypeStruct((n, c1), jnp.float32),
                   jax.ShapeDtypeStruct((n, c1), jnp.float32)],
    )(x, acc0, W_self0, W_neigh0, b0r, wn1p, ws1p)

    acc1 = _make_sc_segsum(n_pad, nch, c1)(y1p, src1, dst1, zeros1)

    out = pl.pallas_call(
        functools.partial(_tc_out_body, ccol=c),
        grid=grid,
        in_specs=[
            pl.BlockSpec((r, c1), lambda i: (i, 0)),
            pl.BlockSpec((NC, r, c1), lambda i: (0, i, 0)),
            pl.BlockSpec((1, c1), lambda i: (0, 0)),
        ],
        out_specs=pl.BlockSpec((r, c1), lambda i: (i, 0)),
        out_shape=jax.ShapeDtypeStruct((n, c1), jnp.float32),
    )(hs, acc1, b1p)

    return out[:, :c]


# speedup vs baseline: 7.4076x; 1.1756x over previous
import functools

import jax
import jax.numpy as jnp
from jax import lax
from jax.experimental import pallas as pl
from jax.experimental.pallas import tpu as pltpu
from jax.experimental.pallas import tpu_sc as plsc

NC = 2
NS = 16
NW = NC * NS
CH = 64


def _round_up(v, m):
    return (v + m - 1) // m * m


@functools.lru_cache(maxsize=None)
def _make_sc_segsum(n_pad, nch, width):
    rows_per_tile = n_pad // NS
    mesh = plsc.VectorSubcoreMesh(
        core_axis_name="c", subcore_axis_name="s", num_cores=NC, num_subcores=NS
    )

    @functools.partial(
        pl.kernel,
        out_type=jax.ShapeDtypeStruct((NC, n_pad, width), jnp.float32),
        mesh=mesh,
        scratch_types=[
            pltpu.VMEM((nch, CH), jnp.int32),
            pltpu.VMEM((nch, CH), jnp.int32),
            pltpu.VMEM((CH, width), jnp.float32),
            pltpu.VMEM((CH, width), jnp.float32),
            pltpu.VMEM_SHARED((n_pad, width), jnp.float32),
            pltpu.SemaphoreType.DMA,
            pltpu.SemaphoreType.DMA,
        ],
        compiler_params=pltpu.CompilerParams(use_tc_tiling_on_sc=False),
    )
    def sc_segsum(table_hbm, src_hbm, dst_hbm, zeros_hbm, out_hbm,
                  src_v, dst_v, rows_a, rows_b, acc_sh, sem_a, sem_b):
        cid = lax.axis_index("c")
        sid = lax.axis_index("s")
        wid = sid * NC + cid
        row0 = sid * rows_per_tile
        pltpu.sync_copy(zeros_hbm, acc_sh.at[pl.ds(row0, rows_per_tile)])
        pltpu.sync_copy(src_hbm.at[wid], src_v)
        pltpu.sync_copy(dst_hbm.at[wid], dst_v)
        plsc.subcore_barrier()

        pltpu.async_copy(table_hbm.at[src_v.at[0]], rows_a, sem_a)

        def body(p, carry):
            j0 = 2 * p
            pltpu.async_copy(table_hbm.at[src_v.at[j0 + 1]], rows_b, sem_b)
            pltpu.make_async_copy(table_hbm.at[src_v.at[j0]], rows_a, sem_a).wait()
            pltpu.sync_copy(rows_a, acc_sh.at[dst_v.at[j0]], add=True)

            @pl.when(j0 + 2 < nch)
            def _():
                pltpu.async_copy(table_hbm.at[src_v.at[j0 + 2]], rows_a, sem_a)

            pltpu.make_async_copy(table_hbm.at[src_v.at[j0 + 1]], rows_b, sem_b).wait()
            pltpu.sync_copy(rows_b, acc_sh.at[dst_v.at[j0 + 1]], add=True)
            return carry

        lax.fori_loop(0, nch // 2, body, 0)
        plsc.subcore_barrier()
        pltpu.sync_copy(acc_sh.at[pl.ds(row0, rows_per_tile)],
                        out_hbm.at[cid, pl.ds(row0, rows_per_tile)])

    return sc_segsum


def _tc_mid_body(x_ref, acc_ref, ws0_ref, wn0_ref, b0_ref, wn1_ref, ws1_ref,
                 y1_ref, hs_ref, *, dcol, c1):
    a = acc_ref[0] + acc_ref[1]
    deg = jnp.clip(a[:, dcol:dcol + 1], 1.0, None)
    mean = a[:, :dcol] / deg
    h = jnp.dot(x_ref[...], ws0_ref[...], preferred_element_type=jnp.float32)
    h = h + jnp.dot(mean, wn0_ref[...], preferred_element_type=jnp.float32)
    h = jnp.maximum(h + b0_ref[...], 0.0)
    y1 = jnp.dot(h, wn1_ref[...], preferred_element_type=jnp.float32)
    col = lax.broadcasted_iota(jnp.int32, y1.shape, 1)
    y1_ref[...] = jnp.where(col == c1 - 1, 1.0, y1)
    hs_ref[...] = jnp.dot(h, ws1_ref[...], preferred_element_type=jnp.float32)


def _tc_out_body(hs_ref, acc_ref, b1_ref, o_ref, *, ccol):
    a = acc_ref[0] + acc_ref[1]
    deg = jnp.clip(a[:, ccol:ccol + 1], 1.0, None)
    o_ref[...] = hs_ref[...] + a / deg + b1_ref[...]


def _prep_edges(edge_index, e_pad, nch, dump_row):
    src = edge_index[0].astype(jnp.int32)
    dst = edge_index[1].astype(jnp.int32)
    pad = e_pad - src.shape[0]
    if pad:
        src = jnp.concatenate([src, jnp.zeros((pad,), jnp.int32)])
        dst = jnp.concatenate([dst, jnp.full((pad,), dump_row, jnp.int32)])
    return src.reshape(NW, nch, CH), dst.reshape(NW, nch, CH)


def kernel(x, edge_index0, edge_index1, W_self0, W_neigh0, b0, W_self1, W_neigh1, b1):
    n, d = x.shape
    h_dim = W_self0.shape[1]
    c = W_self1.shape[1]
    e = edge_index0.shape[1]

    w0 = d + 16
    c1 = _round_up(c + 1, 16)
    n_pad = _round_up(n + 1, 128)
    e_pad = _round_up(e, NW * CH * 2)
    nch = e_pad // (NW * CH)
    rows_per_tile = n_pad // NS

    src0, dst0 = _prep_edges(edge_index0, e_pad, nch, n)
    src1, dst1 = _prep_edges(edge_index1, e_pad, nch, n)

    table0 = jnp.concatenate(
        [x, jnp.ones((n, 1), jnp.float32), jnp.zeros((n, 15), jnp.float32)], axis=1)
    zeros0 = jnp.zeros((rows_per_tile, w0), jnp.float32)
    zeros1 = jnp.zeros((rows_per_tile, c1), jnp.float32)

    b0r = b0.reshape(1, h_dim)
    b1p = jnp.concatenate([b1, jnp.zeros((c1 - c,), jnp.float32)]).reshape(1, c1)
    wn1p = jnp.pad(W_neigh1, ((0, 0), (0, c1 - c)))
    ws1p = jnp.pad(W_self1, ((0, 0), (0, c1 - c)))

    acc0 = _make_sc_segsum(n_pad, nch, w0)(table0, src0, dst0, zeros0)

    r = 1000
    grid = (n // r,)
    y1p, hs = pl.pallas_call(
        functools.partial(_tc_mid_body, dcol=d, c1=c1),
        grid=grid,
        in_specs=[
            pl.BlockSpec((r, d), lambda i: (i, 0)),
            pl.BlockSpec((NC, r, w0), lambda i: (0, i, 0)),
            pl.BlockSpec((d, h_dim), lambda i: (0, 0)),
            pl.BlockSpec((d, h_dim), lambda i: (0, 0)),
            pl.BlockSpec((1, h_dim), lambda i: (0, 0)),
            pl.BlockSpec((h_dim, c1), lambda i: (0, 0)),
            pl.BlockSpec((h_dim, c1), lambda i: (0, 0)),
        ],
        out_specs=[pl.BlockSpec((r, c1), lambda i: (i, 0)),
                   pl.BlockSpec((r, c1), lambda i: (i, 0))],
        out_shape=[jax.ShapeDtypeStruct((n, c1), jnp.float32),
                   jax.ShapeDtypeStruct((n, c1), jnp.float32)],
    )(x, acc0, W_self0, W_neigh0, b0r, wn1p, ws1p)

    acc1 = _make_sc_segsum(n_pad, nch, c1)(y1p, src1, dst1, zeros1)

    out = pl.pallas_call(
        functools.partial(_tc_out_body, ccol=c),
        grid=grid,
        in_specs=[
            pl.BlockSpec((r, c1), lambda i: (i, 0)),
            pl.BlockSpec((NC, r, c1), lambda i: (0, i, 0)),
            pl.BlockSpec((1, c1), lambda i: (0, 0)),
        ],
        out_specs=pl.BlockSpec((r, c1), lambda i: (i, 0)),
        out_shape=jax.ShapeDtypeStruct((n, c1), jnp.float32),
    )(hs, acc1, b1p)

    return out[:, :c]
